# baseline (speedup 1.0000x reference)
import functools

import jax
import jax.numpy as jnp
from jax import lax
from jax.experimental import pallas as pl
from jax.experimental.pallas import tpu as pltpu
from jax.experimental.pallas import tpu_sc as plsc

ALPHA = 0.5
NC = 2
NS = 16
L = 16


def _deg_body(e, ei0_hbm, ei1_hbm, z1d_hbm, deg_hbm, idxv, ones_v, deg_sh, semd):
    c = lax.axis_index("c")
    s = lax.axis_index("s")
    npad = deg_sh.shape[0]
    per = npad // NS
    ept = e // NS
    full = ept // 128
    rem = ept % 128
    pltpu.sync_copy(z1d_hbm, deg_sh.at[pl.ds(s * per, per)])
    for k in range(8):
        ones_v[pl.ds(k * 16, 16)] = jnp.ones((16,), jnp.float32)

    @pl.when(c == 0)
    def _():
        pltpu.sync_copy(ei0_hbm.at[pl.ds(s * ept, ept)], idxv)

    @pl.when(c == 1)
    def _():
        pltpu.sync_copy(ei1_hbm.at[pl.ds(s * ept, ept)], idxv)

    plsc.subcore_barrier()

    def body(j, carry):
        pltpu.async_copy(ones_v, deg_sh.at[idxv.at[pl.ds(j * 128, 128)]],
                         semd, add=True)
        return carry

    lax.fori_loop(0, full, body, 0)
    if rem:
        pltpu.async_copy(ones_v.at[pl.ds(0, rem)],
                         deg_sh.at[idxv.at[pl.ds(full * 128, rem)]],
                         semd, add=True)

    def drain(j, carry):
        pltpu.make_async_copy(ones_v, deg_sh.at[idxv.at[pl.ds(j * 128, 128)]],
                              semd).wait()
        return carry

    lax.fori_loop(0, full, drain, 0)
    if rem:
        pltpu.make_async_copy(ones_v.at[pl.ds(0, rem)],
                              deg_sh.at[idxv.at[pl.ds(full * 128, rem)]],
                              semd).wait()
    plsc.subcore_barrier()
    pltpu.sync_copy(deg_sh.at[pl.ds(s * per, per)],
                    deg_hbm.at[c, pl.ds(s * per, per)])


def _scatter_body(e, v_hbm, ei0_hbm, ei1_hbm, z2d_hbm, t_hbm,
                  idx_g0, idx_g1, idx_s0, idx_s1, rows0, rows1, t_sh,
                  sem0, sem1, semi0, semi1):
    c = lax.axis_index("c")
    s = lax.axis_index("s")
    npad = t_sh.shape[0]
    per = npad // NS
    ept = e // NS
    EB = idx_g0.shape[0]
    nfull = ept // EB
    tail = ept % EB

    def gather(g_ref, k, nrow, rows, sem):
        pltpu.async_copy(
            v_hbm.at[c].at[g_ref.at[pl.ds(k * 128, nrow)]],
            rows.at[pl.ds(0, nrow)], sem)

    def drain_scatter(g_ref, s_ref, k, nrow, rows, sem):
        pltpu.make_async_copy(
            v_hbm.at[c].at[g_ref.at[pl.ds(k * 128, nrow)]],
            rows.at[pl.ds(0, nrow)], sem).wait()
        pltpu.sync_copy(rows.at[pl.ds(0, nrow)],
                        t_sh.at[s_ref.at[pl.ds(k * 128, nrow)]], add=True)

    def load_idx(i, size, g_ref, s_ref, semi):
        off = s * ept + i * EB
        @pl.when(c == 0)
        def _():
            pltpu.async_copy(ei1_hbm.at[pl.ds(off, size)],
                             g_ref.at[pl.ds(0, size)], semi)
            pltpu.async_copy(ei0_hbm.at[pl.ds(off, size)],
                             s_ref.at[pl.ds(0, size)], semi)

        @pl.when(c == 1)
        def _():
            pltpu.async_copy(ei0_hbm.at[pl.ds(off, size)],
                             g_ref.at[pl.ds(0, size)], semi)
            pltpu.async_copy(ei1_hbm.at[pl.ds(off, size)],
                             s_ref.at[pl.ds(0, size)], semi)

    def drain_idx(i, size, g_ref, s_ref, semi):
        off = s * ept + i * EB
        pltpu.make_async_copy(ei0_hbm.at[pl.ds(off, size)],
                              g_ref.at[pl.ds(0, size)], semi).wait()
        pltpu.make_async_copy(ei0_hbm.at[pl.ds(off, size)],
                              s_ref.at[pl.ds(0, size)], semi).wait()

    def run_chunks(g_ref, s_ref, nchunk):
        gather(g_ref, 0, 128, rows0, sem0)

        def pair(jj, carry2):
            j = 2 * jj
            gather(g_ref, j + 1, 128, rows1, sem1)
            drain_scatter(g_ref, s_ref, j, 128, rows0, sem0)

            @pl.when(jj < nchunk // 2 - 1)
            def _():
                gather(g_ref, j + 2, 128, rows0, sem0)

            drain_scatter(g_ref, s_ref, j + 1, 128, rows1, sem1)
            return carry2

        lax.fori_loop(0, nchunk // 2, pair, 0)

    load_idx(0, EB, idx_g0, idx_s0, semi0)
    pltpu.sync_copy(z2d_hbm, t_sh.at[pl.ds(s * per, per)])
    plsc.subcore_barrier()

    def blk(i, carry):
        p = lax.rem(i, 2)

        @pl.when(i + 1 < nfull)
        def _():
            @pl.when(p == 0)
            def _():
                load_idx(i + 1, EB, idx_g1, idx_s1, semi1)

            @pl.when(p == 1)
            def _():
                load_idx(i + 1, EB, idx_g0, idx_s0, semi0)

        @pl.when(p == 0)
        def _():
            drain_idx(i, EB, idx_g0, idx_s0, semi0)
            run_chunks(idx_g0, idx_s0, EB // 128)

        @pl.when(p == 1)
        def _():
            drain_idx(i, EB, idx_g1, idx_s1, semi1)
            run_chunks(idx_g1, idx_s1, EB // 128)

        return carry

    lax.fori_loop(0, nfull, blk, 0)

    if tail:
        load_idx(nfull, tail, idx_g0, idx_s0, semi0)
        drain_idx(nfull, tail, idx_g0, idx_s0, semi0)
        tfull = tail // 128
        trem = tail % 128
        if tfull >= 2:
            run_chunks(idx_g0, idx_s0, (tfull // 2) * 2)
        if tfull % 2:
            gather(idx_g0, tfull - 1, 128, rows0, sem0)
            drain_scatter(idx_g0, idx_s0, tfull - 1, 128, rows0, sem0)
        if trem:
            gather(idx_g0, tfull, trem, rows0, sem0)
            drain_scatter(idx_g0, idx_s0, tfull, trem, rows0, sem0)

    plsc.subcore_barrier()
    pltpu.sync_copy(t_sh.at[pl.ds(s * per, per)],
                    t_hbm.at[c, pl.ds(s * per, per)])


def _matmul_body(x_ref, w1_ref, w2_ref, u_ref):
    xb = x_ref[...]
    u_ref[0] = lax.dot_general(xb, w1_ref[...], (((1,), (1,)), ((), ())),
                               preferred_element_type=jnp.float32)
    u_ref[1] = lax.dot_general(xb, w2_ref[...], (((1,), (1,)), ((), ())),
                               preferred_element_type=jnp.float32)


def _scale_body(u_ref, degs_ref, v_ref, s_ref):
    degs = degs_ref[...]
    inv = jnp.where(degs > 0, lax.rsqrt(degs), 0.0)
    b = inv[:, 0:1]
    a = inv[:, 1:2]
    v_ref[0] = a * u_ref[0]
    v_ref[1] = b * u_ref[1]
    s_ref[...] = jnp.concatenate([ALPHA * b, (1.0 - ALPHA) * a], axis=1)


def _combine_body(t_ref, s_ref, b1_ref, b2_ref, o_ref):
    s = s_ref[...]
    cb = ALPHA * b1_ref[...] + (1.0 - ALPHA) * b2_ref[...]
    o_ref[...] = s[:, 0:1] * t_ref[0] + s[:, 1:2] * t_ref[1] + cb


def kernel(x, edge_index, W_src_to_dst, b_src_to_dst, W_dst_to_src, b_dst_to_src):
    n, d = x.shape
    e = edge_index.shape[1]
    npad = ((n + 511) // 512) * 512
    if npad % (NS * 8):
        npad += NS * 8 - npad % (NS * 8)
    per = npad // NS

    ei0 = edge_index[0].astype(jnp.int32)
    ei1 = edge_index[1].astype(jnp.int32)

    z1d = jnp.zeros((per,), jnp.float32)
    z2d = jnp.zeros((per, d), jnp.float32)

    mesh = plsc.VectorSubcoreMesh(core_axis_name="c", subcore_axis_name="s")

    degs = pl.kernel(
        functools.partial(_deg_body, e),
        out_type=jax.ShapeDtypeStruct((2, npad), jnp.float32),
        mesh=mesh,
        scratch_types=[
            pltpu.VMEM((e // NS,), jnp.int32),
            pltpu.VMEM((128,), jnp.float32),
            pltpu.VMEM_SHARED((npad,), jnp.float32),
            pltpu.SemaphoreType.DMA,
        ],
    )(ei0, ei1, z1d)

    blk = 2048
    grid = npad // blk
    u = pl.pallas_call(
        _matmul_body,
        grid=(grid,),
        in_specs=[
            pl.BlockSpec((blk, d), lambda i: (i, 0)),
            pl.BlockSpec((d, d), lambda i: (0, 0)),
            pl.BlockSpec((d, d), lambda i: (0, 0)),
        ],
        out_specs=pl.BlockSpec((2, blk, d), lambda i: (0, i, 0)),
        out_shape=jax.ShapeDtypeStruct((2, npad, d), jnp.float32),
    )(x, W_src_to_dst, W_dst_to_src)

    v_s = pl.pallas_call(
        _scale_body,
        grid=(grid,),
        in_specs=[
            pl.BlockSpec((2, blk, d), lambda i: (0, i, 0)),
            pl.BlockSpec((blk, 2), lambda i: (i, 0)),
        ],
        out_specs=[
            pl.BlockSpec((2, blk, d), lambda i: (0, i, 0)),
            pl.BlockSpec((blk, 2), lambda i: (i, 0)),
        ],
        out_shape=[
            jax.ShapeDtypeStruct((2, npad, d), jnp.float32),
            jax.ShapeDtypeStruct((npad, 2), jnp.float32),
        ],
    )(u, degs.T)
    v, s_scale = v_s

    t = pl.kernel(
        functools.partial(_scatter_body, e),
        out_type=jax.ShapeDtypeStruct((2, npad, d), jnp.float32),
        mesh=mesh,
        scratch_types=[
            pltpu.VMEM((2048,), jnp.int32),
            pltpu.VMEM((2048,), jnp.int32),
            pltpu.VMEM((2048,), jnp.int32),
            pltpu.VMEM((2048,), jnp.int32),
            pltpu.VMEM((128, d), jnp.float32),
            pltpu.VMEM((128, d), jnp.float32),
            pltpu.VMEM_SHARED((npad, d), jnp.float32),
            pltpu.SemaphoreType.DMA,
            pltpu.SemaphoreType.DMA,
            pltpu.SemaphoreType.DMA,
            pltpu.SemaphoreType.DMA,
        ],
    )(v, ei0, ei1, z2d)

    out = pl.pallas_call(
        _combine_body,
        grid=(grid,),
        in_specs=[
            pl.BlockSpec((2, blk, d), lambda i: (0, i, 0)),
            pl.BlockSpec((blk, 2), lambda i: (i, 0)),
            pl.BlockSpec((1, d), lambda i: (0, 0)),
            pl.BlockSpec((1, d), lambda i: (0, 0)),
        ],
        out_specs=pl.BlockSpec((blk, d), lambda i: (i, 0)),
        out_shape=jax.ShapeDtypeStruct((n, d), jnp.float32),
    )(t, s_scale, b_src_to_dst.reshape(1, d), b_dst_to_src.reshape(1, d))

    return out


# speedup vs baseline: 1.0116x; 1.0116x over previous
import functools

import jax
import jax.numpy as jnp
from jax import lax
from jax.experimental import pallas as pl
from jax.experimental.pallas import tpu as pltpu
from jax.experimental.pallas import tpu_sc as plsc

ALPHA = 0.5
NC = 2
NS = 16
L = 16


def _deg_body(e, ei0_hbm, ei1_hbm, z1d_hbm, deg_hbm, idxv, ones_v, deg_sh, semd):
    c = lax.axis_index("c")
    s = lax.axis_index("s")
    npad = deg_sh.shape[0]
    per = npad // NS
    ept = e // NS
    full = ept // 128
    rem = ept % 128
    pltpu.sync_copy(z1d_hbm, deg_sh.at[pl.ds(s * per, per)])
    for k in range(8):
        ones_v[pl.ds(k * 16, 16)] = jnp.ones((16,), jnp.float32)

    @pl.when(c == 0)
    def _():
        pltpu.sync_copy(ei0_hbm.at[pl.ds(s * ept, ept)], idxv)

    @pl.when(c == 1)
    def _():
        pltpu.sync_copy(ei1_hbm.at[pl.ds(s * ept, ept)], idxv)

    plsc.subcore_barrier()

    def body(j, carry):
        pltpu.async_copy(ones_v, deg_sh.at[idxv.at[pl.ds(j * 128, 128)]],
                         semd, add=True)
        return carry

    lax.fori_loop(0, full, body, 0)
    if rem:
        pltpu.async_copy(ones_v.at[pl.ds(0, rem)],
                         deg_sh.at[idxv.at[pl.ds(full * 128, rem)]],
                         semd, add=True)

    def drain(j, carry):
        pltpu.make_async_copy(ones_v, deg_sh.at[idxv.at[pl.ds(j * 128, 128)]],
                              semd).wait()
        return carry

    lax.fori_loop(0, full, drain, 0)
    if rem:
        pltpu.make_async_copy(ones_v.at[pl.ds(0, rem)],
                              deg_sh.at[idxv.at[pl.ds(full * 128, rem)]],
                              semd).wait()
    plsc.subcore_barrier()
    pltpu.sync_copy(deg_sh.at[pl.ds(s * per, per)],
                    deg_hbm.at[c, pl.ds(s * per, per)])


def _scatter_body(e, v_hbm, ei0_hbm, ei1_hbm, z2d_hbm, t_hbm,
                  idx_g0, idx_g1, idx_s0, idx_s1, rows0, rows1, t_sh,
                  sem0, sem1, semi0, semi1):
    c = lax.axis_index("c")
    s = lax.axis_index("s")
    npad = t_sh.shape[0]
    per = npad // NS
    ept = e // NS
    EB = idx_g0.shape[0]
    nfull = ept // EB
    tail = ept % EB

    def gather(g_ref, k, nrow, rows, sem):
        pltpu.async_copy(
            v_hbm.at[c].at[g_ref.at[pl.ds(k * 128, nrow)]],
            rows.at[pl.ds(0, nrow)], sem)

    def drain_scatter(g_ref, s_ref, k, nrow, rows, sem):
        pltpu.make_async_copy(
            v_hbm.at[c].at[g_ref.at[pl.ds(k * 128, nrow)]],
            rows.at[pl.ds(0, nrow)], sem).wait()
        pltpu.sync_copy(rows.at[pl.ds(0, nrow)],
                        t_sh.at[s_ref.at[pl.ds(k * 128, nrow)]], add=True)

    def load_idx(i, size, g_ref, s_ref, semi):
        off = s * ept + i * EB
        @pl.when(c == 0)
        def _():
            pltpu.async_copy(ei1_hbm.at[pl.ds(off, size)],
                             g_ref.at[pl.ds(0, size)], semi)
            pltpu.async_copy(ei0_hbm.at[pl.ds(off, size)],
                             s_ref.at[pl.ds(0, size)], semi)

        @pl.when(c == 1)
        def _():
            pltpu.async_copy(ei0_hbm.at[pl.ds(off, size)],
                             g_ref.at[pl.ds(0, size)], semi)
            pltpu.async_copy(ei1_hbm.at[pl.ds(off, size)],
                             s_ref.at[pl.ds(0, size)], semi)

    def drain_idx(i, size, g_ref, s_ref, semi):
        off = s * ept + i * EB
        pltpu.make_async_copy(ei0_hbm.at[pl.ds(off, size)],
                              g_ref.at[pl.ds(0, size)], semi).wait()
        pltpu.make_async_copy(ei0_hbm.at[pl.ds(off, size)],
                              s_ref.at[pl.ds(0, size)], semi).wait()

    def run_chunks(g_ref, s_ref, nchunk):
        gather(g_ref, 0, 128, rows0, sem0)

        def pair(jj, carry2):
            j = 2 * jj
            gather(g_ref, j + 1, 128, rows1, sem1)
            drain_scatter(g_ref, s_ref, j, 128, rows0, sem0)

            @pl.when(jj < nchunk // 2 - 1)
            def _():
                gather(g_ref, j + 2, 128, rows0, sem0)

            drain_scatter(g_ref, s_ref, j + 1, 128, rows1, sem1)
            return carry2

        lax.fori_loop(0, nchunk // 2, pair, 0)

    load_idx(0, EB, idx_g0, idx_s0, semi0)
    pltpu.sync_copy(z2d_hbm, t_sh.at[pl.ds(s * per, per)])
    plsc.subcore_barrier()

    def blk(i, carry):
        p = lax.rem(i, 2)

        @pl.when(i + 1 < nfull)
        def _():
            @pl.when(p == 0)
            def _():
                load_idx(i + 1, EB, idx_g1, idx_s1, semi1)

            @pl.when(p == 1)
            def _():
                load_idx(i + 1, EB, idx_g0, idx_s0, semi0)

        @pl.when(p == 0)
        def _():
            drain_idx(i, EB, idx_g0, idx_s0, semi0)
            run_chunks(idx_g0, idx_s0, EB // 128)

        @pl.when(p == 1)
        def _():
            drain_idx(i, EB, idx_g1, idx_s1, semi1)
            run_chunks(idx_g1, idx_s1, EB // 128)

        return carry

    lax.fori_loop(0, nfull, blk, 0)

    if tail:
        load_idx(nfull, tail, idx_g0, idx_s0, semi0)
        drain_idx(nfull, tail, idx_g0, idx_s0, semi0)
        tfull = tail // 128
        trem = tail % 128
        if tfull >= 2:
            run_chunks(idx_g0, idx_s0, (tfull // 2) * 2)
        if tfull % 2:
            gather(idx_g0, tfull - 1, 128, rows0, sem0)
            drain_scatter(idx_g0, idx_s0, tfull - 1, 128, rows0, sem0)
        if trem:
            gather(idx_g0, tfull, trem, rows0, sem0)
            drain_scatter(idx_g0, idx_s0, tfull, trem, rows0, sem0)

    plsc.subcore_barrier()
    pltpu.sync_copy(t_sh.at[pl.ds(s * per, per)],
                    t_hbm.at[c, pl.ds(s * per, per)])


def _prescale_body(x_ref, w1_ref, w2_ref, degs_ref, v_ref, s_ref):
    xb = x_ref[...]
    u1 = lax.dot_general(xb, w1_ref[...], (((1,), (1,)), ((), ())),
                         preferred_element_type=jnp.float32)
    u2 = lax.dot_general(xb, w2_ref[...], (((1,), (1,)), ((), ())),
                         preferred_element_type=jnp.float32)
    degs = degs_ref[...]
    inv = jnp.where(degs > 0, lax.rsqrt(degs), 0.0)
    b = inv[:, 0:1]
    a = inv[:, 1:2]
    v_ref[0] = a * u1
    v_ref[1] = b * u2
    s_ref[...] = jnp.concatenate([ALPHA * b, (1.0 - ALPHA) * a], axis=1)


def _combine_body(t_ref, s_ref, b1_ref, b2_ref, o_ref):
    s = s_ref[...]
    cb = ALPHA * b1_ref[...] + (1.0 - ALPHA) * b2_ref[...]
    o_ref[...] = s[:, 0:1] * t_ref[0] + s[:, 1:2] * t_ref[1] + cb


def kernel(x, edge_index, W_src_to_dst, b_src_to_dst, W_dst_to_src, b_dst_to_src):
    n, d = x.shape
    e = edge_index.shape[1]
    npad = ((n + 511) // 512) * 512
    if npad % (NS * 8):
        npad += NS * 8 - npad % (NS * 8)
    per = npad // NS

    ei0 = edge_index[0].astype(jnp.int32)
    ei1 = edge_index[1].astype(jnp.int32)

    z1d = jnp.zeros((per,), jnp.float32)
    z2d = jnp.zeros((per, d), jnp.float32)

    mesh = plsc.VectorSubcoreMesh(core_axis_name="c", subcore_axis_name="s")

    degs = pl.kernel(
        functools.partial(_deg_body, e),
        out_type=jax.ShapeDtypeStruct((2, npad), jnp.float32),
        mesh=mesh,
        scratch_types=[
            pltpu.VMEM((e // NS,), jnp.int32),
            pltpu.VMEM((128,), jnp.float32),
            pltpu.VMEM_SHARED((npad,), jnp.float32),
            pltpu.SemaphoreType.DMA,
        ],
    )(ei0, ei1, z1d)

    blk = 2048
    grid = npad // blk
    v_s = pl.pallas_call(
        _prescale_body,
        grid=(grid,),
        in_specs=[
            pl.BlockSpec((blk, d), lambda i: (i, 0)),
            pl.BlockSpec((d, d), lambda i: (0, 0)),
            pl.BlockSpec((d, d), lambda i: (0, 0)),
            pl.BlockSpec((blk, 2), lambda i: (i, 0)),
        ],
        out_specs=[
            pl.BlockSpec((2, blk, d), lambda i: (0, i, 0)),
            pl.BlockSpec((blk, 2), lambda i: (i, 0)),
        ],
        out_shape=[
            jax.ShapeDtypeStruct((2, npad, d), jnp.float32),
            jax.ShapeDtypeStruct((npad, 2), jnp.float32),
        ],
    )(x, W_src_to_dst, W_dst_to_src, degs.T)
    v, s_scale = v_s

    t = pl.kernel(
        functools.partial(_scatter_body, e),
        out_type=jax.ShapeDtypeStruct((2, npad, d), jnp.float32),
        mesh=mesh,
        scratch_types=[
            pltpu.VMEM((2048,), jnp.int32),
            pltpu.VMEM((2048,), jnp.int32),
            pltpu.VMEM((2048,), jnp.int32),
            pltpu.VMEM((2048,), jnp.int32),
            pltpu.VMEM((128, d), jnp.float32),
            pltpu.VMEM((128, d), jnp.float32),
            pltpu.VMEM_SHARED((npad, d), jnp.float32),
            pltpu.SemaphoreType.DMA,
            pltpu.SemaphoreType.DMA,
            pltpu.SemaphoreType.DMA,
            pltpu.SemaphoreType.DMA,
        ],
    )(v, ei0, ei1, z2d)

    out = pl.pallas_call(
        _combine_body,
        grid=(grid,),
        in_specs=[
            pl.BlockSpec((2, blk, d), lambda i: (0, i, 0)),
            pl.BlockSpec((blk, 2), lambda i: (i, 0)),
            pl.BlockSpec((1, d), lambda i: (0, 0)),
            pl.BlockSpec((1, d), lambda i: (0, 0)),
        ],
        out_specs=pl.BlockSpec((blk, d), lambda i: (i, 0)),
        out_shape=jax.ShapeDtypeStruct((n, d), jnp.float32),
    )(t, s_scale, b_src_to_dst.reshape(1, d), b_dst_to_src.reshape(1, d))

    return out
